# baseline (speedup 1.0000x reference)
import jax
import jax.numpy as jnp
from jax import lax
from jax.experimental import pallas as pl
from jax.experimental.pallas import tpu as pltpu
from jax.experimental.pallas import tpu_sc as plsc

_N_TABLE = 1000
_B = 16384
_NC = 1
_NS = 16
_NW = _NC * _NS
_B_PER_W = _B // _NW
_CHUNK = 128
_N_CHUNKS = _B_PER_W // _CHUNK


def _gather_body(t_hbm, table_hbm, out_hbm, idx_v, vals_v, sem):
    wid = lax.axis_index("s")
    pltpu.sync_copy(t_hbm.at[wid], idx_v)
    copies = [
        pltpu.async_copy(
            table_hbm.at[idx_v.at[j]],
            vals_v.at[j],
            sem,
        )
        for j in range(_N_CHUNKS)
    ]
    for cp in copies:
        cp.wait()
    pltpu.sync_copy(vals_v, out_hbm.at[wid])


_gather = pl.kernel(
    _gather_body,
    out_type=jax.ShapeDtypeStruct((_NW, _N_CHUNKS, _CHUNK), jnp.float32),
    mesh=plsc.VectorSubcoreMesh(
        core_axis_name="c", subcore_axis_name="s", num_cores=_NC
    ),
    scratch_types=[
        pltpu.VMEM((_N_CHUNKS, _CHUNK), jnp.int32),
        pltpu.VMEM((_N_CHUNKS, _CHUNK), jnp.float32),
        pltpu.SemaphoreType.DMA,
    ],
    compiler_params=pltpu.CompilerParams(needs_layout_passes=False),
)


@jax.jit
def kernel(t, beta_schedule):
    t3 = t.astype(jnp.int32).reshape(_NW, _N_CHUNKS, _CHUNK)
    return _gather(t3, beta_schedule).reshape(_B)


# speedup vs baseline: 1.4765x; 1.4765x over previous
import jax
import jax.numpy as jnp
from jax import lax
from jax.experimental import pallas as pl
from jax.experimental.pallas import tpu as pltpu
from jax.experimental.pallas import tpu_sc as plsc

_N_TABLE = 1000
_B = 16384
_NC = 1
_NS = 16
_NW = _NC * _NS
_L = 16
_B_PER_W = _B // _NW
_H = _B_PER_W // 2


def _gather_body(
    t_hbm, table_hbm, out_hbm, idx_v, vals_v, tab_v, sem_t, sem_a, sem_b
):
    wid = lax.axis_index("s")
    base = wid * _B_PER_W
    cp_tab = pltpu.async_copy(table_hbm, tab_v, sem_t)
    cp_i0 = pltpu.async_copy(t_hbm.at[pl.ds(base, _H)], idx_v.at[pl.ds(0, _H)], sem_a)
    cp_i1 = pltpu.async_copy(
        t_hbm.at[pl.ds(base + _H, _H)], idx_v.at[pl.ds(_H, _H)], sem_b
    )
    cp_tab.wait()
    cp_i0.wait()

    def step(i, carry):
        idx = idx_v[pl.ds(i * _L, _L)]
        vals_v[pl.ds(i * _L, _L)] = plsc.load_gather(tab_v, [idx])
        return carry

    lax.fori_loop(0, _H // _L, step, 0, unroll=4)
    cp_o0 = pltpu.async_copy(
        vals_v.at[pl.ds(0, _H)], out_hbm.at[pl.ds(base, _H)], sem_a
    )
    cp_i1.wait()
    lax.fori_loop(_H // _L, _B_PER_W // _L, step, 0, unroll=4)
    cp_o1 = pltpu.async_copy(
        vals_v.at[pl.ds(_H, _H)], out_hbm.at[pl.ds(base + _H, _H)], sem_b
    )
    cp_o0.wait()
    cp_o1.wait()


_gather = pl.kernel(
    _gather_body,
    out_type=jax.ShapeDtypeStruct((_B,), jnp.float32),
    mesh=plsc.VectorSubcoreMesh(
        core_axis_name="c", subcore_axis_name="s", num_cores=_NC
    ),
    scratch_types=[
        pltpu.VMEM((_B_PER_W,), jnp.int32),
        pltpu.VMEM((_B_PER_W,), jnp.float32),
        pltpu.VMEM((_N_TABLE,), jnp.float32),
        pltpu.SemaphoreType.DMA,
        pltpu.SemaphoreType.DMA,
        pltpu.SemaphoreType.DMA,
    ],
    compiler_params=pltpu.CompilerParams(needs_layout_passes=False),
)


@jax.jit
def kernel(t, beta_schedule):
    return _gather(t.astype(jnp.int32), beta_schedule)


# speedup vs baseline: 1.4870x; 1.0071x over previous
import functools

import jax
import jax.numpy as jnp
from jax import lax
from jax.experimental import pallas as pl
from jax.experimental.pallas import tpu as pltpu
from jax.experimental.pallas import tpu_sc as plsc

_N_TABLE = 1000
_B = 16384
_NC = 1
_NS = 16
_NW = _NC * _NS
_L = 16
_B_PER_W = _B // _NW


def _gather_body(t_hbm, table_hbm, out_hbm, idx_v, vals_v, tab_v, sem_t, sem_i):
    wid = lax.axis_index("s") * _NC + lax.axis_index("c")
    base = wid * _B_PER_W
    cp_tab = pltpu.async_copy(table_hbm, tab_v, sem_t)
    cp_idx = pltpu.async_copy(t_hbm.at[pl.ds(base, _B_PER_W)], idx_v, sem_i)
    cp_tab.wait()
    cp_idx.wait()

    def step(i, carry):
        idx = idx_v[pl.ds(i * _L, _L)]
        vals_v[pl.ds(i * _L, _L)] = plsc.load_gather(tab_v, [idx])
        return carry

    lax.fori_loop(0, _B_PER_W // _L, step, 0, unroll=4)
    pltpu.sync_copy(vals_v, out_hbm.at[pl.ds(base, _B_PER_W)])


_gather = pl.kernel(
    _gather_body,
    out_type=jax.ShapeDtypeStruct((_B,), jnp.float32),
    mesh=plsc.VectorSubcoreMesh(
        core_axis_name="c", subcore_axis_name="s", num_cores=_NC
    ),
    scratch_types=[
        pltpu.VMEM((_B_PER_W,), jnp.int32),
        pltpu.VMEM((_B_PER_W,), jnp.float32),
        pltpu.VMEM((_N_TABLE,), jnp.float32),
        pltpu.SemaphoreType.DMA,
        pltpu.SemaphoreType.DMA,
    ],
    compiler_params=pltpu.CompilerParams(
        needs_layout_passes=False,
        disable_bounds_checks=True,
        disable_semaphore_checks=True,
        skip_device_barrier=True,
    ),
)


@jax.jit
def kernel(t, beta_schedule):
    return _gather(t.astype(jnp.int32), beta_schedule)
